# baseline (speedup 1.0000x reference)
import math

import numpy as np
import jax
import jax.numpy as jnp
from jax import lax
from jax.experimental import pallas as pl
from jax.experimental.pallas import tpu as pltpu

_ALPHA = 1.9
_BETA = 3.8
_GAMMA = 15.2
_BI = 3
_S = 7
_NB = 50
_DEPTH = 12
_H = 12
_EMB = 512
_MAXH = 14
_PH = 12
_B = 32
_LK = 72
_L = 74
_DH = _DEPTH * _H
_NIDS = 3 * _PH * _PH
_IC = 4
_NSTEP = 19
_LPAD = _IC * _NSTEP
_SPAD = 80
_NBP = _NB


def _piecewise_np(rp):
    rpf = rp.astype(np.float32)
    rp_abs = np.abs(rpf)
    log_den = math.log(_GAMMA / _ALPHA)
    y = np.sign(rpf) * np.minimum(
        np.round(_ALPHA + np.log(np.maximum(rp_abs, 1e-6) / _ALPHA) / log_den * (_BETA - _ALPHA)),
        np.float32(_BETA),
    )
    idx = np.where(rp_abs <= _ALPHA, rpf, y)
    return idx.astype(np.int32)


def _build_t432():
    r = np.arange(_MAXH, dtype=np.int32)
    ap = np.stack(np.meshgrid(r, r, indexing="ij"), axis=-1).reshape(-1, 2)
    diff = ap[:, None, :] - ap[None, :, :]
    t196 = (_piecewise_np(diff[..., 0]) + _BI) * _S + (_piecewise_np(diff[..., 1]) + _BI)

    p = np.arange(_PH, dtype=np.float32)
    pos = np.stack(np.meshgrid(p, p, indexing="ij"), axis=-1).reshape(-1, 2) / (_PH - 1)
    full = np.concatenate([pos, pos, pos], axis=0)
    ldmks = full * 2.0 - 1.0
    x = ldmks[:, 0]
    y = ldmks[:, 1]
    ix = np.clip(np.round((x + 1.0) * 0.5 * (_MAXH - 1)).astype(np.int32), 0, _MAXH - 1)
    iy = np.clip(np.round((y + 1.0) * 0.5 * (_MAXH - 1)).astype(np.int32), 0, _MAXH - 1)
    g = iy * _MAXH + ix
    return t196[g[:, None], g[None, :]].astype(np.float32)


_T432 = _build_t432()


def _kern(et_ref, ids_ref, w_ref, b_ref, t_ref, *refs):
    out_refs = refs[:_DEPTH]
    mi_s = refs[_DEPTH]
    wb_s = refs[_DEPTH + 1]
    t = pl.program_id(0)

    @pl.when(t == 0)
    def _():
        wb_s[...] = w_ref[...].astype(jnp.bfloat16)
        ids = ids_ref[:, 0, :]
        oh = (ids[:, :, None] ==
              lax.broadcasted_iota(jnp.int32, (_B, _LK, _NIDS), 2)
              ).astype(jnp.bfloat16)
        sel = lax.dot_general(oh.reshape(_B * _LK, _NIDS),
                              t_ref[...].astype(jnp.bfloat16),
                              (((1,), (0,)), ((), ())),
                              preferred_element_type=jnp.float32)
        sel = sel.reshape(_B, _LK, _NIDS).astype(jnp.bfloat16)
        core = lax.dot_general(sel, oh, (((2,), (2,)), ((0,), (0,))),
                               preferred_element_type=jnp.float32)
        padv = jnp.float32(_NB - 1)
        cpad = jnp.full((_B, _LK, 1), padv, jnp.float32)
        rpad = jnp.full((_B, 1, _L), padv, jnp.float32)
        mf = jnp.concatenate(
            [rpad, jnp.concatenate([cpad, core, cpad], axis=2), rpad], axis=1
        )
        mi = mf.astype(jnp.int32).transpose(1, 0, 2)
        mi_s[0:_L] = mi
        mi_s[_L:_SPAD] = jnp.full((_SPAD - _L, _B, _L), _NB - 1, jnp.int32)

    ec = et_ref[...].reshape(_IC * _B, _EMB)
    a = lax.dot_general(ec, wb_s[...], (((1,), (1,)), ((), ())),
                        preferred_element_type=jnp.float32)
    a = a + b_ref[0][None, :]
    a3 = a.astype(jnp.bfloat16).reshape(_IC * _B, _DH, _NBP)

    mi_c = mi_s[pl.ds(t * _IC, _IC)].reshape(_IC * _B, _L)
    ohb = (mi_c[:, :, None] ==
           lax.broadcasted_iota(jnp.int32, (_IC * _B, _L, _NBP), 2)
           ).astype(jnp.bfloat16)
    res = lax.dot_general(a3, ohb, (((2,), (2,)), ((0,), (0,))),
                          preferred_element_type=jnp.float32)
    rt = res.astype(jnp.bfloat16).reshape(_IC, _B, _DH, _L).transpose(2, 0, 1, 3)
    for d in range(_DEPTH):
        out_refs[d][...] = rt[d * _H:(d + 1) * _H].astype(jnp.float32)


def kernel(rel_kp_embs, ids_keep, W, b):
    tf = jnp.asarray(_T432)
    ids3 = ids_keep.reshape(_B, 1, _LK)
    b2 = b.reshape(1, _DH * _NBP)
    et = jnp.pad(
        rel_kp_embs.astype(jnp.bfloat16).transpose(1, 0, 2),
        ((0, _LPAD - _L), (0, 0), (0, 0)),
    )
    wb = W.T
    outs = pl.pallas_call(
        _kern,
        grid=(_NSTEP,),
        in_specs=[
            pl.BlockSpec((_IC, _B, _EMB), lambda t: (t, 0, 0)),
            pl.BlockSpec((_B, 1, _LK), lambda t: (0, 0, 0)),
            pl.BlockSpec((_DH * _NBP, _EMB), lambda t: (0, 0)),
            pl.BlockSpec((1, _DH * _NBP), lambda t: (0, 0)),
            pl.BlockSpec((_NIDS, _NIDS), lambda t: (0, 0)),
        ],
        out_specs=[
            pl.BlockSpec((_H, _IC, _B, _L), lambda t: (0, t, 0, 0))
            for _ in range(_DEPTH)
        ],
        out_shape=[
            jax.ShapeDtypeStruct((_H, _L, _B, _L), jnp.float32)
            for _ in range(_DEPTH)
        ],
        scratch_shapes=[
            pltpu.VMEM((_SPAD, _B, _L), jnp.int32),
            pltpu.VMEM((_DH * _NBP, _EMB), jnp.bfloat16),
        ],
        compiler_params=pltpu.CompilerParams(
            dimension_semantics=("arbitrary",),
        ),
    )(et, ids3, wb, b2, tf)
    return tuple(jnp.transpose(o, (2, 0, 1, 3)) for o in outs)


# speedup vs baseline: 1.1779x; 1.1779x over previous
import math

import numpy as np
import jax
import jax.numpy as jnp
from jax import lax
from jax.experimental import pallas as pl
from jax.experimental.pallas import tpu as pltpu

_ALPHA = 1.9
_BETA = 3.8
_GAMMA = 15.2
_BI = 3
_S = 7
_NB = 50
_DEPTH = 12
_H = 12
_EMB = 512
_MAXH = 14
_PH = 12
_B = 32
_LK = 72
_L = 74
_DH = _DEPTH * _H
_NIDS = 3 * _PH * _PH
_IC = 4
_NSTEP = 19
_LPAD = _IC * _NSTEP
_SPAD = 80
_NBP = _NB


def _piecewise_np(rp):
    rpf = rp.astype(np.float32)
    rp_abs = np.abs(rpf)
    log_den = math.log(_GAMMA / _ALPHA)
    y = np.sign(rpf) * np.minimum(
        np.round(_ALPHA + np.log(np.maximum(rp_abs, 1e-6) / _ALPHA) / log_den * (_BETA - _ALPHA)),
        np.float32(_BETA),
    )
    idx = np.where(rp_abs <= _ALPHA, rpf, y)
    return idx.astype(np.int32)


def _build_t432():
    r = np.arange(_MAXH, dtype=np.int32)
    ap = np.stack(np.meshgrid(r, r, indexing="ij"), axis=-1).reshape(-1, 2)
    diff = ap[:, None, :] - ap[None, :, :]
    t196 = (_piecewise_np(diff[..., 0]) + _BI) * _S + (_piecewise_np(diff[..., 1]) + _BI)

    p = np.arange(_PH, dtype=np.float32)
    pos = np.stack(np.meshgrid(p, p, indexing="ij"), axis=-1).reshape(-1, 2) / (_PH - 1)
    full = np.concatenate([pos, pos, pos], axis=0)
    ldmks = full * 2.0 - 1.0
    x = ldmks[:, 0]
    y = ldmks[:, 1]
    ix = np.clip(np.round((x + 1.0) * 0.5 * (_MAXH - 1)).astype(np.int32), 0, _MAXH - 1)
    iy = np.clip(np.round((y + 1.0) * 0.5 * (_MAXH - 1)).astype(np.int32), 0, _MAXH - 1)
    g = iy * _MAXH + ix
    return t196[g[:, None], g[None, :]].astype(np.float32)


_T432 = _build_t432()


def _kern(et_ref, ids_ref, w_ref, b_ref, t_ref, *refs):
    out_refs = refs[:_DEPTH]
    mi_s = refs[_DEPTH]
    wb_s = refs[_DEPTH + 1]
    t = pl.program_id(0)

    @pl.when(t == 0)
    def _():
        wb_s[...] = w_ref[...].astype(jnp.bfloat16).transpose(1, 0)
        ids = ids_ref[:, 0, :]
        oh = (ids[:, :, None] ==
              lax.broadcasted_iota(jnp.int32, (_B, _LK, _NIDS), 2)
              ).astype(jnp.bfloat16)
        sel = lax.dot_general(oh.reshape(_B * _LK, _NIDS),
                              t_ref[...].astype(jnp.bfloat16),
                              (((1,), (0,)), ((), ())),
                              preferred_element_type=jnp.float32)
        sel = sel.reshape(_B, _LK, _NIDS).astype(jnp.bfloat16)
        core = lax.dot_general(sel, oh, (((2,), (2,)), ((0,), (0,))),
                               preferred_element_type=jnp.float32)
        padv = jnp.float32(_NB - 1)
        cpad = jnp.full((_B, _LK, 1), padv, jnp.float32)
        rpad = jnp.full((_B, 1, _L), padv, jnp.float32)
        mf = jnp.concatenate(
            [rpad, jnp.concatenate([cpad, core, cpad], axis=2), rpad], axis=1
        )
        mi = mf.astype(jnp.int32).transpose(1, 0, 2)
        mi_s[0:_L] = mi
        mi_s[_L:_SPAD] = jnp.full((_SPAD - _L, _B, _L), _NB - 1, jnp.int32)

    ec = et_ref[...].reshape(_IC * _B, _EMB)
    a = lax.dot_general(ec, wb_s[...], (((1,), (0,)), ((), ())),
                        preferred_element_type=jnp.float32)
    a = a + b_ref[0][None, :]
    a3 = a.astype(jnp.bfloat16).reshape(_IC * _B, _DH, _NBP)

    mi_c = mi_s[pl.ds(t * _IC, _IC)].reshape(_IC * _B, _L)
    ohb = (mi_c[:, :, None] ==
           lax.broadcasted_iota(jnp.int32, (_IC * _B, _L, _NBP), 2)
           ).astype(jnp.bfloat16)
    res = lax.dot_general(a3, ohb, (((2,), (2,)), ((0,), (0,))),
                          preferred_element_type=jnp.float32)
    rt = res.astype(jnp.bfloat16).reshape(_IC, _B, _DH, _L).transpose(2, 0, 1, 3)
    for d in range(_DEPTH):
        out_refs[d][...] = rt[d * _H:(d + 1) * _H].astype(jnp.float32)


def kernel(rel_kp_embs, ids_keep, W, b):
    tf = jnp.asarray(_T432)
    ids3 = ids_keep.reshape(_B, 1, _LK)
    b2 = b.reshape(1, _DH * _NBP)
    et = jnp.pad(
        rel_kp_embs.astype(jnp.bfloat16).transpose(1, 0, 2),
        ((0, _LPAD - _L), (0, 0), (0, 0)),
    )
    wb = W.T
    outs = pl.pallas_call(
        _kern,
        grid=(_NSTEP,),
        in_specs=[
            pl.BlockSpec((_IC, _B, _EMB), lambda t: (t, 0, 0)),
            pl.BlockSpec((_B, 1, _LK), lambda t: (0, 0, 0)),
            pl.BlockSpec((_DH * _NBP, _EMB), lambda t: (0, 0)),
            pl.BlockSpec((1, _DH * _NBP), lambda t: (0, 0)),
            pl.BlockSpec((_NIDS, _NIDS), lambda t: (0, 0)),
        ],
        out_specs=[
            pl.BlockSpec((_H, _IC, _B, _L), lambda t: (0, t, 0, 0))
            for _ in range(_DEPTH)
        ],
        out_shape=[
            jax.ShapeDtypeStruct((_H, _L, _B, _L), jnp.float32)
            for _ in range(_DEPTH)
        ],
        scratch_shapes=[
            pltpu.VMEM((_SPAD, _B, _L), jnp.int32),
            pltpu.VMEM((_EMB, _DH * _NBP), jnp.bfloat16),
        ],
        compiler_params=pltpu.CompilerParams(
            dimension_semantics=("arbitrary",),
        ),
    )(et, ids3, wb, b2, tf)
    return tuple(jnp.transpose(o, (2, 0, 1, 3)) for o in outs)


# speedup vs baseline: 1.1813x; 1.0029x over previous
import math

import numpy as np
import jax
import jax.numpy as jnp
from jax import lax
from jax.experimental import pallas as pl
from jax.experimental.pallas import tpu as pltpu

_ALPHA = 1.9
_BETA = 3.8
_GAMMA = 15.2
_BI = 3
_S = 7
_NB = 50
_DEPTH = 12
_H = 12
_EMB = 512
_MAXH = 14
_PH = 12
_B = 32
_LK = 72
_L = 74
_DH = _DEPTH * _H
_NIDS = 3 * _PH * _PH
_IC = 4
_NSTEP = 19
_LPAD = _IC * _NSTEP
_SPAD = 80
_NBP = _NB


def _piecewise_np(rp):
    rpf = rp.astype(np.float32)
    rp_abs = np.abs(rpf)
    log_den = math.log(_GAMMA / _ALPHA)
    y = np.sign(rpf) * np.minimum(
        np.round(_ALPHA + np.log(np.maximum(rp_abs, 1e-6) / _ALPHA) / log_den * (_BETA - _ALPHA)),
        np.float32(_BETA),
    )
    idx = np.where(rp_abs <= _ALPHA, rpf, y)
    return idx.astype(np.int32)


def _build_t432():
    r = np.arange(_MAXH, dtype=np.int32)
    ap = np.stack(np.meshgrid(r, r, indexing="ij"), axis=-1).reshape(-1, 2)
    diff = ap[:, None, :] - ap[None, :, :]
    t196 = (_piecewise_np(diff[..., 0]) + _BI) * _S + (_piecewise_np(diff[..., 1]) + _BI)

    p = np.arange(_PH, dtype=np.float32)
    pos = np.stack(np.meshgrid(p, p, indexing="ij"), axis=-1).reshape(-1, 2) / (_PH - 1)
    full = np.concatenate([pos, pos, pos], axis=0)
    ldmks = full * 2.0 - 1.0
    x = ldmks[:, 0]
    y = ldmks[:, 1]
    ix = np.clip(np.round((x + 1.0) * 0.5 * (_MAXH - 1)).astype(np.int32), 0, _MAXH - 1)
    iy = np.clip(np.round((y + 1.0) * 0.5 * (_MAXH - 1)).astype(np.int32), 0, _MAXH - 1)
    g = iy * _MAXH + ix
    return t196[g[:, None], g[None, :]].astype(np.float32)


_T432 = _build_t432()


def _kern(et_ref, ids_ref, w_ref, b_ref, t_ref, *refs):
    out_refs = refs[:_DEPTH]
    mi_s = refs[_DEPTH]
    wb_s = refs[_DEPTH + 1]
    t = pl.program_id(0)

    @pl.when(t == 0)
    def _():
        wb_s[...] = w_ref[...].astype(jnp.bfloat16).transpose(1, 0)
        ids = ids_ref[:, 0, :]
        oh = (ids[:, :, None] ==
              lax.broadcasted_iota(jnp.int32, (_B, _LK, _NIDS), 2)
              ).astype(jnp.bfloat16)
        sel = lax.dot_general(oh.reshape(_B * _LK, _NIDS),
                              t_ref[...].astype(jnp.bfloat16),
                              (((1,), (0,)), ((), ())),
                              preferred_element_type=jnp.float32)
        sel = sel.reshape(_B, _LK, _NIDS).astype(jnp.bfloat16)
        core = lax.dot_general(sel, oh, (((2,), (2,)), ((0,), (0,))),
                               preferred_element_type=jnp.float32)
        padv = jnp.float32(_NB - 1)
        cpad = jnp.full((_B, _LK, 1), padv, jnp.float32)
        rpad = jnp.full((_B, 1, _L), padv, jnp.float32)
        mf = jnp.concatenate(
            [rpad, jnp.concatenate([cpad, core, cpad], axis=2), rpad], axis=1
        )
        mi = mf.astype(jnp.int32).transpose(1, 0, 2)
        mi_s[0:_L * _B] = mi.reshape(_L * _B, _L)
        mi_s[_L * _B:] = jnp.full(((_SPAD - _L) * _B, _L), _NB - 1, jnp.int32)

    ec = et_ref[...].reshape(_IC * _B, _EMB)
    a = lax.dot_general(ec, wb_s[...], (((1,), (0,)), ((), ())),
                        preferred_element_type=jnp.float32)
    a = a + b_ref[0][None, :]
    a3 = a.astype(jnp.bfloat16).reshape(_IC * _B, _DH, _NBP)

    mi_c = mi_s[pl.ds(t * _IC * _B, _IC * _B)]
    ohb = (mi_c[:, :, None] ==
           lax.broadcasted_iota(jnp.int32, (_IC * _B, _L, _NBP), 2)
           ).astype(jnp.bfloat16)
    res = lax.dot_general(a3, ohb, (((2,), (2,)), ((0,), (0,))),
                          preferred_element_type=jnp.float32)
    rt = res.astype(jnp.bfloat16).reshape(_IC, _B, _DH, _L).transpose(2, 0, 1, 3)
    for d in range(_DEPTH):
        out_refs[d][...] = rt[d * _H:(d + 1) * _H].astype(jnp.float32)


def kernel(rel_kp_embs, ids_keep, W, b):
    tf = jnp.asarray(_T432)
    ids3 = ids_keep.reshape(_B, 1, _LK)
    b2 = b.reshape(1, _DH * _NBP)
    et = jnp.pad(
        rel_kp_embs.astype(jnp.bfloat16).transpose(1, 0, 2),
        ((0, _LPAD - _L), (0, 0), (0, 0)),
    )
    wb = W.T
    outs = pl.pallas_call(
        _kern,
        grid=(_NSTEP,),
        in_specs=[
            pl.BlockSpec((_IC, _B, _EMB), lambda t: (t, 0, 0)),
            pl.BlockSpec((_B, 1, _LK), lambda t: (0, 0, 0)),
            pl.BlockSpec((_DH * _NBP, _EMB), lambda t: (0, 0)),
            pl.BlockSpec((1, _DH * _NBP), lambda t: (0, 0)),
            pl.BlockSpec((_NIDS, _NIDS), lambda t: (0, 0)),
        ],
        out_specs=[
            pl.BlockSpec((_H, _IC, _B, _L), lambda t: (0, t, 0, 0))
            for _ in range(_DEPTH)
        ],
        out_shape=[
            jax.ShapeDtypeStruct((_H, _L, _B, _L), jnp.float32)
            for _ in range(_DEPTH)
        ],
        scratch_shapes=[
            pltpu.VMEM((_SPAD * _B, _L), jnp.int32),
            pltpu.VMEM((_EMB, _DH * _NBP), jnp.bfloat16),
        ],
        compiler_params=pltpu.CompilerParams(
            dimension_semantics=("arbitrary",),
        ),
    )(et, ids3, wb, b2, tf)
    return tuple(jnp.transpose(o, (2, 0, 1, 3)) for o in outs)
